# Initial kernel scaffold; baseline (speedup 1.0000x reference)
import jax
import jax.numpy as jnp
from jax.experimental import pallas as pl


def kernel(nodes_feat, edge_index, edges_feat, nodes_num_norm_sqrt, edges_num_norm_sqrt, graph_ids, W_emb, b_emb, W1, b1, gamma1, beta1, Wo, bo, gamma_o, beta_o, W_out, b_out):
    raise NotImplementedError("write your pallas kernel here")


# speedup vs baseline: 4.7776x; 4.7776x over previous
import jax
import jax.numpy as jnp
from jax import lax
from jax.experimental import pallas as pl
from jax.experimental.pallas import tpu as pltpu
from jax.experimental.pallas import tpu_sc as plsc

N = 50000
E = 800000
G = 256
D = 64
Q = 16
EPS = 1e-5

NP = 50176
ROWS = 6272
EP = ROWS * 128
TROWS = ROWS // 16
GRP = 8
NGRP = TROWS // GRP
DEGSPLIT = 24
R = NP
ZCH = 784
NT = NP // 16
GACC = 264
PGRPS = NP // 1024
BLK = 3136
GRID = NP // BLK

_mesh = plsc.VectorSubcoreMesh(core_axis_name="c", subcore_axis_name="s")
_sc_params = pltpu.CompilerParams(use_tc_tiling_on_sc=False)


def _make_edge():
  def body(src_hbm, dst_hbm, x_hbm, z16,
           msg_out,
           src_v, dst_v, rows_v, acc_sh, sem):
    c = lax.axis_index("c")
    s = lax.axis_index("s")

    base = s * NT
    for q in range(4):
      pltpu.sync_copy(z16, acc_sh.at[pl.ds(base + q * ZCH, ZCH), :])
    plsc.subcore_barrier()

    @pl.loop(0, NGRP)
    def _(g):
      r0 = s * TROWS + g * GRP
      pltpu.sync_copy(src_hbm.at[c, pl.ds(r0, GRP), :], src_v)
      pltpu.sync_copy(dst_hbm.at[pl.ds(r0, GRP), :], dst_v)
      descs = [pltpu.async_copy(x_hbm.at[src_v.at[j]], rows_v.at[j], sem)
               for j in range(GRP)]
      for j in range(GRP):
        descs[j].wait()
        pltpu.sync_copy(rows_v.at[j], acc_sh.at[dst_v.at[j]], add=True)

    plsc.subcore_barrier()
    pltpu.sync_copy(acc_sh.at[pl.ds(s * NT, NT), :],
                    msg_out.at[c, pl.ds(s * NT, NT), :])

  return pl.kernel(
      body,
      out_type=jax.ShapeDtypeStruct((2, NP, Q), jnp.float32),
      mesh=_mesh,
      scratch_types=[
          pltpu.VMEM((GRP, 128), jnp.int32),
          pltpu.VMEM((GRP, 128), jnp.int32),
          pltpu.VMEM((GRP, 128, Q), jnp.float32),
          pltpu.VMEM_SHARED((R, Q), jnp.float32),
          pltpu.SemaphoreType.DMA,
      ],
      compiler_params=_sc_params)


_edge = _make_edge()


def _make_deg():
  def body(dst_hbm, z8, ones_hbm, deg_out, dst_v, ones_v, deg_sh):
    c = lax.axis_index("c")
    s = lax.axis_index("s")
    base = s * NT
    for q in range(4):
      pltpu.sync_copy(z8, deg_sh.at[pl.ds(base + q * ZCH, ZCH), :])
    pltpu.sync_copy(ones_hbm, ones_v)
    plsc.subcore_barrier()

    @pl.loop(0, NGRP)
    def _(g):
      @pl.when((g < DEGSPLIT) == (c == 0))
      def _():
        r0 = s * TROWS + g * GRP
        pltpu.sync_copy(dst_hbm.at[pl.ds(r0, GRP), :], dst_v)
        for j in range(GRP):
          pltpu.sync_copy(ones_v, deg_sh.at[dst_v.at[j]], add=True)

    plsc.subcore_barrier()
    pltpu.sync_copy(deg_sh.at[pl.ds(s * NT, NT), :],
                    deg_out.at[c, pl.ds(s * NT, NT), :])

  return pl.kernel(
      body,
      out_type=jax.ShapeDtypeStruct((2, NP, 8), jnp.float32),
      mesh=_mesh,
      scratch_types=[
          pltpu.VMEM((GRP, 128), jnp.int32),
          pltpu.VMEM((128, 8), jnp.float32),
          pltpu.VMEM_SHARED((R, 8), jnp.float32),
      ],
      compiler_params=_sc_params)


_deg = _make_deg()


def _make_pool():
  def body(e1_hbm, e2_hbm, gid_hbm, z16, z8, ones_hbm,
           hg_out, cnt_out,
           gid_v, chunk_v, ones_v, acca_sh, accb_sh, cnt_sh):
    c = lax.axis_index("c")
    s = lax.axis_index("s")
    pltpu.sync_copy(z16.at[pl.ds(0, 16), :], acca_sh.at[pl.ds(s * 16, 16), :])
    pltpu.sync_copy(z16.at[pl.ds(0, 16), :], accb_sh.at[pl.ds(s * 16, 16), :])
    pltpu.sync_copy(z8.at[pl.ds(0, 16), :], cnt_sh.at[pl.ds(s * 16, 16), :])
    @pl.when(s == 0)
    def _():
      pltpu.sync_copy(z16.at[pl.ds(16, 8), :], acca_sh.at[pl.ds(G, 8), :])
      pltpu.sync_copy(z16.at[pl.ds(16, 8), :], accb_sh.at[pl.ds(G, 8), :])
      pltpu.sync_copy(z8.at[pl.ds(16, 8), :], cnt_sh.at[pl.ds(G, 8), :])
    pltpu.sync_copy(ones_hbm, ones_v)
    plsc.subcore_barrier()

    @pl.loop(0, (PGRPS + 15) // 16)
    def _(k):
      grp = s + 16 * k
      @pl.when(grp < PGRPS)
      def _():
        pltpu.sync_copy(gid_hbm.at[pl.ds(grp * GRP, GRP), :], gid_v)
        for r in range(GRP):
          na = (2 * c) * NP + grp * 1024 + r * 128
          nb = (2 * c + 1) * NP + grp * 1024 + r * 128
          for e_hbm in (e1_hbm, e2_hbm):
            pltpu.sync_copy(e_hbm.at[pl.ds(na, 128), :], chunk_v)
            pltpu.sync_copy(chunk_v, acca_sh.at[gid_v.at[r]], add=True)
            pltpu.sync_copy(e_hbm.at[pl.ds(nb, 128), :], chunk_v)
            pltpu.sync_copy(chunk_v, accb_sh.at[gid_v.at[r]], add=True)
          @pl.when(c == 0)
          def _():
            pltpu.sync_copy(ones_v, cnt_sh.at[gid_v.at[r]], add=True)

    plsc.subcore_barrier()
    pltpu.sync_copy(acca_sh.at[pl.ds(s * 16, 16), :],
                    hg_out.at[2 * c, pl.ds(s * 16, 16), :])
    pltpu.sync_copy(accb_sh.at[pl.ds(s * 16, 16), :],
                    hg_out.at[2 * c + 1, pl.ds(s * 16, 16), :])
    @pl.when(c == 0)
    def _():
      pltpu.sync_copy(cnt_sh.at[pl.ds(s * 16, 16), :],
                      cnt_out.at[pl.ds(s * 16, 16), :])

  return pl.kernel(
      body,
      out_type=[jax.ShapeDtypeStruct((4, G, Q), jnp.float32),
                jax.ShapeDtypeStruct((G, 8), jnp.float32)],
      mesh=_mesh,
      scratch_types=[
          pltpu.VMEM((GRP, 128), jnp.int32),
          pltpu.VMEM((128, Q), jnp.float32),
          pltpu.VMEM((128, 8), jnp.float32),
          pltpu.VMEM_SHARED((GACC, Q), jnp.float32),
          pltpu.VMEM_SHARED((GACC, Q), jnp.float32),
          pltpu.VMEM_SHARED((GACC, 8), jnp.float32),
      ],
      compiler_params=_sc_params)


_pool = _make_pool()


def _quarter_out_specs():
  return pl.BlockSpec((4, BLK, Q), lambda i: (0, i, 0))


def _quarter_in_specs():
  return [pl.BlockSpec((1, BLK, Q), lambda i, q=q: (q, i, 0))
          for q in range(4)]


def _emb_call(x, w, b):
  def body(x_ref, w_ref, b_ref, o_ref):
    y = jnp.dot(x_ref[...], w_ref[...],
                preferred_element_type=jnp.float32) + b_ref[...]
    for q in range(4):
      o_ref[q, :, :] = y[:, q * Q:(q + 1) * Q]

  return pl.pallas_call(
      body,
      grid=(GRID,),
      in_specs=[pl.BlockSpec((BLK, D), lambda i: (i, 0)),
                pl.BlockSpec((D, D), lambda i: (0, 0)),
                pl.BlockSpec((1, D), lambda i: (0, 0))],
      out_specs=_quarter_out_specs(),
      out_shape=jax.ShapeDtypeStruct((4, NP, Q), jnp.float32),
  )(x, w, b)


def _layer_a(msgA, msgB, deg2, snorm, w, b):
  def body(m0, m1, m2, m3, d0, d1, sn, w_ref, b_ref, o_ref, s_ref, ss_ref):
    deg = d0[0, :, 0:1] + d1[0, :, 0:1]
    inv = 1.0 / jnp.maximum(deg, 1.0)
    h = b_ref[...] * jnp.ones((BLK, 1), jnp.float32)
    for q, m in enumerate((m0, m1, m2, m3)):
      h = h + jnp.dot(m[0] * inv, w_ref[q * Q:(q + 1) * Q, :],
                      preferred_element_type=jnp.float32)
    h = jnp.maximum(h, 0.0) * sn[...]
    for q in range(4):
      o_ref[q, :, :] = h[:, q * Q:(q + 1) * Q]

    @pl.when(pl.program_id(0) == 0)
    def _():
      s_ref[...] = jnp.zeros_like(s_ref)
      ss_ref[...] = jnp.zeros_like(ss_ref)

    rowid = (lax.broadcasted_iota(jnp.int32, (BLK, 1), 0)
             + pl.program_id(0) * BLK)
    hm = jnp.where(rowid < N, h, 0.0)
    s_ref[...] += jnp.sum(hm, axis=0, keepdims=True)
    ss_ref[...] += jnp.sum(hm * hm, axis=0, keepdims=True)

  qa = [pl.BlockSpec((1, BLK, Q), lambda i, c=c: (c, i, 0)) for c in range(2)]
  return pl.pallas_call(
      body,
      grid=(GRID,),
      in_specs=qa + qa +
      [pl.BlockSpec((1, BLK, 8), lambda i: (0, i, 0)),
       pl.BlockSpec((1, BLK, 8), lambda i: (1, i, 0)),
       pl.BlockSpec((BLK, 1), lambda i: (i, 0)),
       pl.BlockSpec((D, D), lambda i: (0, 0)),
       pl.BlockSpec((1, D), lambda i: (0, 0))],
      out_specs=[_quarter_out_specs(),
                 pl.BlockSpec((1, D), lambda i: (0, 0)),
                 pl.BlockSpec((1, D), lambda i: (0, 0))],
      out_shape=[jax.ShapeDtypeStruct((4, NP, Q), jnp.float32),
                 jax.ShapeDtypeStruct((1, D), jnp.float32),
                 jax.ShapeDtypeStruct((1, D), jnp.float32)],
  )(msgA, msgA, msgB, msgB, deg2, deg2, snorm, w, b)


def _layer_b(hF, xF, ssum, sqsum, gamma, beta):
  def body(h0, h1, h2, h3, x0, x1, x2, x3, s_ref, ss_ref, g_ref, be_ref,
           o_ref):
    mu = s_ref[...] / N
    var = ss_ref[...] / N - mu * mu
    scale = g_ref[...] * lax.rsqrt(var + EPS)
    shift = be_ref[...] - mu * scale
    hs = (h0, h1, h2, h3)
    xs = (x0, x1, x2, x3)
    for q in range(4):
      sl = slice(q * Q, (q + 1) * Q)
      o_ref[q, :, :] = xs[q][0] + hs[q][0] * scale[:, sl] + shift[:, sl]

  return pl.pallas_call(
      body,
      grid=(GRID,),
      in_specs=_quarter_in_specs() + _quarter_in_specs() +
      [pl.BlockSpec((1, D), lambda i: (0, 0)),
       pl.BlockSpec((1, D), lambda i: (0, 0)),
       pl.BlockSpec((1, D), lambda i: (0, 0)),
       pl.BlockSpec((1, D), lambda i: (0, 0))],
      out_specs=_quarter_out_specs(),
      out_shape=jax.ShapeDtypeStruct((4, NP, Q), jnp.float32),
  )(hF, hF, hF, hF, xF, xF, xF, xF, ssum, sqsum, gamma, beta)


def _final(hgsum, cnt, w, b):
  def body(s0, s1, s2, s3, c_ref, w_ref, b_ref, o_ref):
    inv = 1.0 / jnp.maximum(c_ref[:, 0:1], 1.0)
    o = b_ref[...] * jnp.ones((G, 1), jnp.float32)
    for q, sq in enumerate((s0, s1, s2, s3)):
      o = o + jnp.dot(sq[0] * inv, w_ref[q * Q:(q + 1) * Q, :],
                      preferred_element_type=jnp.float32)
    o_ref[...] = o

  return pl.pallas_call(
      body,
      grid=(1,),
      in_specs=[pl.BlockSpec((1, G, Q), lambda i, q=q: (q, 0, 0))
                for q in range(4)] +
      [pl.BlockSpec((G, 8), lambda i: (0, 0)),
       pl.BlockSpec((D, D), lambda i: (0, 0)),
       pl.BlockSpec((1, D), lambda i: (0, 0))],
      out_specs=pl.BlockSpec((G, D), lambda i: (0, 0)),
      out_shape=jax.ShapeDtypeStruct((G, D), jnp.float32),
  )(hgsum, hgsum, hgsum, hgsum, cnt, w, b)


def kernel(nodes_feat, edge_index, edges_feat, nodes_num_norm_sqrt,
           edges_num_norm_sqrt, graph_ids, W_emb, b_emb, W1, b1, gamma1,
           beta1, Wo, bo, gamma_o, beta_o, W_out, b_out):
  src = edge_index[0]
  dst = edge_index[1]
  pad = EP - E
  srcP = jnp.concatenate([src, jnp.zeros((pad,), jnp.int32)])
  dstP = jnp.concatenate([dst, jnp.full((pad,), N, jnp.int32)])
  srcA = jnp.stack([srcP, srcP + NP]).reshape(2, ROWS, 128)
  srcB = jnp.stack([srcP + 2 * NP, srcP + 3 * NP]).reshape(2, ROWS, 128)
  dst2 = dstP.reshape(ROWS, 128)
  gidP = jnp.concatenate([graph_ids,
                          jnp.full((NP - N,), G, jnp.int32)]).reshape(
                              PGRPS * GRP, 128)
  z16 = jnp.zeros((ZCH, Q), jnp.float32)
  z8 = jnp.zeros((ZCH, 8), jnp.float32)
  ones128 = jnp.ones((128, 8), jnp.float32)

  deg2 = _deg(dst2, z8, ones128)
  e0F = _emb_call(nodes_feat, W_emb, b_emb.reshape(1, D))
  e0flat = e0F.reshape(4 * NP, Q)
  msg1A = _edge(srcA, dst2, e0flat, z16)
  msg1B = _edge(srcB, dst2, e0flat, z16)
  h1F, s1, ss1 = _layer_a(msg1A, msg1B, deg2, nodes_num_norm_sqrt, W1,
                          b1.reshape(1, D))
  e1F = _layer_b(h1F, e0F, s1, ss1, gamma1.reshape(1, D),
                 beta1.reshape(1, D))
  e1flat = e1F.reshape(4 * NP, Q)
  msg2A = _edge(srcA, dst2, e1flat, z16)
  msg2B = _edge(srcB, dst2, e1flat, z16)
  h2F, s2, ss2 = _layer_a(msg2A, msg2B, deg2, nodes_num_norm_sqrt, Wo,
                          bo.reshape(1, D))
  e2F = _layer_b(h2F, e1F, s2, ss2, gamma_o.reshape(1, D),
                 beta_o.reshape(1, D))
  hgsum, cnt = _pool(e1flat, e2F.reshape(4 * NP, Q), gidP, z16, z8, ones128)
  return _final(hgsum, cnt, W_out, b_out.reshape(1, D))


# speedup vs baseline: 5.4749x; 1.1460x over previous
import jax
import jax.numpy as jnp
from jax import lax
from jax.experimental import pallas as pl
from jax.experimental.pallas import tpu as pltpu
from jax.experimental.pallas import tpu_sc as plsc

N = 50000
E = 800000
G = 256
D = 64
Q = 16
EPS = 1e-5

NP = 50176
ROWS = 6272
EP = ROWS * 128
TROWS = ROWS // 16
GRP = 8
NGRP = TROWS // GRP
DEGSPLIT = 24
R = NP
ZCH = 784
NT = NP // 16
GACC = 264
PGRPS = NP // 1024
BLK = 3136
GRID = NP // BLK

_mesh = plsc.VectorSubcoreMesh(core_axis_name="c", subcore_axis_name="s")
_sc_params = pltpu.CompilerParams(use_tc_tiling_on_sc=False)


def _make_edge():
  def body(src_hbm, dst_hbm, x_hbm, z16,
           msg_out,
           src_a, dst_a, rows_a, src_b, dst_b, rows_b,
           acc_sh, sem_a, sem_b):
    c = lax.axis_index("c")
    s = lax.axis_index("s")

    base = s * NT
    for q in range(4):
      pltpu.sync_copy(z16, acc_sh.at[pl.ds(base + q * ZCH, ZCH), :])
    plsc.subcore_barrier()

    def fire(g, src_v, dst_v, rows_v, sem):
      r0 = s * TROWS + g * GRP
      pltpu.sync_copy(src_hbm.at[c, pl.ds(r0, GRP), :], src_v)
      pltpu.sync_copy(dst_hbm.at[pl.ds(r0, GRP), :], dst_v)
      for j in range(GRP):
        pltpu.async_copy(x_hbm.at[src_v.at[j]], rows_v.at[j], sem)

    def drain_scatter(src_v, dst_v, rows_v, sem):
      for j in range(GRP):
        pltpu.make_async_copy(x_hbm.at[src_v.at[j]], rows_v.at[j],
                              sem).wait()
        pltpu.sync_copy(rows_v.at[j], acc_sh.at[dst_v.at[j]], add=True)

    fire(0, src_a, dst_a, rows_a, sem_a)

    @pl.loop(0, NGRP // 2)
    def _(k):
      fire(2 * k + 1, src_b, dst_b, rows_b, sem_b)
      drain_scatter(src_a, dst_a, rows_a, sem_a)
      fire(2 * k + 2, src_a, dst_a, rows_a, sem_a)
      drain_scatter(src_b, dst_b, rows_b, sem_b)

    drain_scatter(src_a, dst_a, rows_a, sem_a)

    plsc.subcore_barrier()
    pltpu.sync_copy(acc_sh.at[pl.ds(s * NT, NT), :],
                    msg_out.at[c, pl.ds(s * NT, NT), :])

  return pl.kernel(
      body,
      out_type=jax.ShapeDtypeStruct((2, NP, Q), jnp.float32),
      mesh=_mesh,
      scratch_types=[
          pltpu.VMEM((GRP, 128), jnp.int32),
          pltpu.VMEM((GRP, 128), jnp.int32),
          pltpu.VMEM((GRP, 128, Q), jnp.float32),
          pltpu.VMEM((GRP, 128), jnp.int32),
          pltpu.VMEM((GRP, 128), jnp.int32),
          pltpu.VMEM((GRP, 128, Q), jnp.float32),
          pltpu.VMEM_SHARED((R, Q), jnp.float32),
          pltpu.SemaphoreType.DMA,
          pltpu.SemaphoreType.DMA,
      ],
      compiler_params=_sc_params)


_edge = _make_edge()


def _make_deg():
  def body(dst_hbm, z8, ones_hbm, deg_out, dst_v, ones_v, deg_sh):
    c = lax.axis_index("c")
    s = lax.axis_index("s")
    base = s * NT
    for q in range(4):
      pltpu.sync_copy(z8, deg_sh.at[pl.ds(base + q * ZCH, ZCH), :])
    pltpu.sync_copy(ones_hbm, ones_v)
    plsc.subcore_barrier()

    @pl.loop(0, NGRP)
    def _(g):
      @pl.when((g < DEGSPLIT) == (c == 0))
      def _():
        r0 = s * TROWS + g * GRP
        pltpu.sync_copy(dst_hbm.at[pl.ds(r0, GRP), :], dst_v)
        for j in range(GRP):
          pltpu.sync_copy(ones_v, deg_sh.at[dst_v.at[j]], add=True)

    plsc.subcore_barrier()
    pltpu.sync_copy(deg_sh.at[pl.ds(s * NT, NT), :],
                    deg_out.at[c, pl.ds(s * NT, NT), :])

  return pl.kernel(
      body,
      out_type=jax.ShapeDtypeStruct((2, NP, 8), jnp.float32),
      mesh=_mesh,
      scratch_types=[
          pltpu.VMEM((GRP, 128), jnp.int32),
          pltpu.VMEM((128, 8), jnp.float32),
          pltpu.VMEM_SHARED((R, 8), jnp.float32),
      ],
      compiler_params=_sc_params)


_deg = _make_deg()


def _make_pool():
  def body(e1_hbm, e2_hbm, gid_hbm, z16, z8, ones_hbm,
           hg_out, cnt_out,
           gid_v, chunk_v, ones_v, acca_sh, accb_sh, cnt_sh):
    c = lax.axis_index("c")
    s = lax.axis_index("s")
    pltpu.sync_copy(z16.at[pl.ds(0, 16), :], acca_sh.at[pl.ds(s * 16, 16), :])
    pltpu.sync_copy(z16.at[pl.ds(0, 16), :], accb_sh.at[pl.ds(s * 16, 16), :])
    pltpu.sync_copy(z8.at[pl.ds(0, 16), :], cnt_sh.at[pl.ds(s * 16, 16), :])
    @pl.when(s == 0)
    def _():
      pltpu.sync_copy(z16.at[pl.ds(16, 8), :], acca_sh.at[pl.ds(G, 8), :])
      pltpu.sync_copy(z16.at[pl.ds(16, 8), :], accb_sh.at[pl.ds(G, 8), :])
      pltpu.sync_copy(z8.at[pl.ds(16, 8), :], cnt_sh.at[pl.ds(G, 8), :])
    pltpu.sync_copy(ones_hbm, ones_v)
    plsc.subcore_barrier()

    @pl.loop(0, (PGRPS + 15) // 16)
    def _(k):
      grp = s + 16 * k
      @pl.when(grp < PGRPS)
      def _():
        pltpu.sync_copy(gid_hbm.at[pl.ds(grp * GRP, GRP), :], gid_v)
        for r in range(GRP):
          na = (2 * c) * NP + grp * 1024 + r * 128
          nb = (2 * c + 1) * NP + grp * 1024 + r * 128
          for e_hbm in (e1_hbm, e2_hbm):
            pltpu.sync_copy(e_hbm.at[pl.ds(na, 128), :], chunk_v)
            pltpu.sync_copy(chunk_v, acca_sh.at[gid_v.at[r]], add=True)
            pltpu.sync_copy(e_hbm.at[pl.ds(nb, 128), :], chunk_v)
            pltpu.sync_copy(chunk_v, accb_sh.at[gid_v.at[r]], add=True)
          @pl.when(c == 0)
          def _():
            pltpu.sync_copy(ones_v, cnt_sh.at[gid_v.at[r]], add=True)

    plsc.subcore_barrier()
    pltpu.sync_copy(acca_sh.at[pl.ds(s * 16, 16), :],
                    hg_out.at[2 * c, pl.ds(s * 16, 16), :])
    pltpu.sync_copy(accb_sh.at[pl.ds(s * 16, 16), :],
                    hg_out.at[2 * c + 1, pl.ds(s * 16, 16), :])
    @pl.when(c == 0)
    def _():
      pltpu.sync_copy(cnt_sh.at[pl.ds(s * 16, 16), :],
                      cnt_out.at[pl.ds(s * 16, 16), :])

  return pl.kernel(
      body,
      out_type=[jax.ShapeDtypeStruct((4, G, Q), jnp.float32),
                jax.ShapeDtypeStruct((G, 8), jnp.float32)],
      mesh=_mesh,
      scratch_types=[
          pltpu.VMEM((GRP, 128), jnp.int32),
          pltpu.VMEM((128, Q), jnp.float32),
          pltpu.VMEM((128, 8), jnp.float32),
          pltpu.VMEM_SHARED((GACC, Q), jnp.float32),
          pltpu.VMEM_SHARED((GACC, Q), jnp.float32),
          pltpu.VMEM_SHARED((GACC, 8), jnp.float32),
      ],
      compiler_params=_sc_params)


_pool = _make_pool()


def _quarter_out_specs():
  return pl.BlockSpec((4, BLK, Q), lambda i: (0, i, 0))


def _quarter_in_specs():
  return [pl.BlockSpec((1, BLK, Q), lambda i, q=q: (q, i, 0))
          for q in range(4)]


def _emb_call(x, w, b):
  def body(x_ref, w_ref, b_ref, o_ref):
    y = jnp.dot(x_ref[...], w_ref[...],
                preferred_element_type=jnp.float32) + b_ref[...]
    for q in range(4):
      o_ref[q, :, :] = y[:, q * Q:(q + 1) * Q]

  return pl.pallas_call(
      body,
      grid=(GRID,),
      in_specs=[pl.BlockSpec((BLK, D), lambda i: (i, 0)),
                pl.BlockSpec((D, D), lambda i: (0, 0)),
                pl.BlockSpec((1, D), lambda i: (0, 0))],
      out_specs=_quarter_out_specs(),
      out_shape=jax.ShapeDtypeStruct((4, NP, Q), jnp.float32),
  )(x, w, b)


def _layer_a(msgA, msgB, deg2, snorm, w, b):
  def body(m0, m1, m2, m3, d0, d1, sn, w_ref, b_ref, o_ref, s_ref, ss_ref):
    deg = d0[0, :, 0:1] + d1[0, :, 0:1]
    inv = 1.0 / jnp.maximum(deg, 1.0)
    h = b_ref[...] * jnp.ones((BLK, 1), jnp.float32)
    for q, m in enumerate((m0, m1, m2, m3)):
      h = h + jnp.dot(m[0] * inv, w_ref[q * Q:(q + 1) * Q, :],
                      preferred_element_type=jnp.float32)
    h = jnp.maximum(h, 0.0) * sn[...]
    for q in range(4):
      o_ref[q, :, :] = h[:, q * Q:(q + 1) * Q]

    @pl.when(pl.program_id(0) == 0)
    def _():
      s_ref[...] = jnp.zeros_like(s_ref)
      ss_ref[...] = jnp.zeros_like(ss_ref)

    rowid = (lax.broadcasted_iota(jnp.int32, (BLK, 1), 0)
             + pl.program_id(0) * BLK)
    hm = jnp.where(rowid < N, h, 0.0)
    s_ref[...] += jnp.sum(hm, axis=0, keepdims=True)
    ss_ref[...] += jnp.sum(hm * hm, axis=0, keepdims=True)

  qa = [pl.BlockSpec((1, BLK, Q), lambda i, c=c: (c, i, 0)) for c in range(2)]
  return pl.pallas_call(
      body,
      grid=(GRID,),
      in_specs=qa + qa +
      [pl.BlockSpec((1, BLK, 8), lambda i: (0, i, 0)),
       pl.BlockSpec((1, BLK, 8), lambda i: (1, i, 0)),
       pl.BlockSpec((BLK, 1), lambda i: (i, 0)),
       pl.BlockSpec((D, D), lambda i: (0, 0)),
       pl.BlockSpec((1, D), lambda i: (0, 0))],
      out_specs=[_quarter_out_specs(),
                 pl.BlockSpec((1, D), lambda i: (0, 0)),
                 pl.BlockSpec((1, D), lambda i: (0, 0))],
      out_shape=[jax.ShapeDtypeStruct((4, NP, Q), jnp.float32),
                 jax.ShapeDtypeStruct((1, D), jnp.float32),
                 jax.ShapeDtypeStruct((1, D), jnp.float32)],
  )(msgA, msgA, msgB, msgB, deg2, deg2, snorm, w, b)


def _layer_b(hF, xF, ssum, sqsum, gamma, beta):
  def body(h0, h1, h2, h3, x0, x1, x2, x3, s_ref, ss_ref, g_ref, be_ref,
           o_ref):
    mu = s_ref[...] / N
    var = ss_ref[...] / N - mu * mu
    scale = g_ref[...] * lax.rsqrt(var + EPS)
    shift = be_ref[...] - mu * scale
    hs = (h0, h1, h2, h3)
    xs = (x0, x1, x2, x3)
    for q in range(4):
      sl = slice(q * Q, (q + 1) * Q)
      o_ref[q, :, :] = xs[q][0] + hs[q][0] * scale[:, sl] + shift[:, sl]

  return pl.pallas_call(
      body,
      grid=(GRID,),
      in_specs=_quarter_in_specs() + _quarter_in_specs() +
      [pl.BlockSpec((1, D), lambda i: (0, 0)),
       pl.BlockSpec((1, D), lambda i: (0, 0)),
       pl.BlockSpec((1, D), lambda i: (0, 0)),
       pl.BlockSpec((1, D), lambda i: (0, 0))],
      out_specs=_quarter_out_specs(),
      out_shape=jax.ShapeDtypeStruct((4, NP, Q), jnp.float32),
  )(hF, hF, hF, hF, xF, xF, xF, xF, ssum, sqsum, gamma, beta)


def _final(hgsum, cnt, w, b):
  def body(s0, s1, s2, s3, c_ref, w_ref, b_ref, o_ref):
    inv = 1.0 / jnp.maximum(c_ref[:, 0:1], 1.0)
    o = b_ref[...] * jnp.ones((G, 1), jnp.float32)
    for q, sq in enumerate((s0, s1, s2, s3)):
      o = o + jnp.dot(sq[0] * inv, w_ref[q * Q:(q + 1) * Q, :],
                      preferred_element_type=jnp.float32)
    o_ref[...] = o

  return pl.pallas_call(
      body,
      grid=(1,),
      in_specs=[pl.BlockSpec((1, G, Q), lambda i, q=q: (q, 0, 0))
                for q in range(4)] +
      [pl.BlockSpec((G, 8), lambda i: (0, 0)),
       pl.BlockSpec((D, D), lambda i: (0, 0)),
       pl.BlockSpec((1, D), lambda i: (0, 0))],
      out_specs=pl.BlockSpec((G, D), lambda i: (0, 0)),
      out_shape=jax.ShapeDtypeStruct((G, D), jnp.float32),
  )(hgsum, hgsum, hgsum, hgsum, cnt, w, b)


def kernel(nodes_feat, edge_index, edges_feat, nodes_num_norm_sqrt,
           edges_num_norm_sqrt, graph_ids, W_emb, b_emb, W1, b1, gamma1,
           beta1, Wo, bo, gamma_o, beta_o, W_out, b_out):
  src = edge_index[0]
  dst = edge_index[1]
  pad = EP - E
  srcP = jnp.concatenate([src, jnp.zeros((pad,), jnp.int32)])
  dstP = jnp.concatenate([dst, jnp.full((pad,), N, jnp.int32)])
  srcA = jnp.stack([srcP, srcP + NP]).reshape(2, ROWS, 128)
  srcB = jnp.stack([srcP + 2 * NP, srcP + 3 * NP]).reshape(2, ROWS, 128)
  dst2 = dstP.reshape(ROWS, 128)
  gidP = jnp.concatenate([graph_ids,
                          jnp.full((NP - N,), G, jnp.int32)]).reshape(
                              PGRPS * GRP, 128)
  z16 = jnp.zeros((ZCH, Q), jnp.float32)
  z8 = jnp.zeros((ZCH, 8), jnp.float32)
  ones128 = jnp.ones((128, 8), jnp.float32)

  deg2 = _deg(dst2, z8, ones128)
  e0F = _emb_call(nodes_feat, W_emb, b_emb.reshape(1, D))
  e0flat = e0F.reshape(4 * NP, Q)
  msg1A = _edge(srcA, dst2, e0flat, z16)
  msg1B = _edge(srcB, dst2, e0flat, z16)
  h1F, s1, ss1 = _layer_a(msg1A, msg1B, deg2, nodes_num_norm_sqrt, W1,
                          b1.reshape(1, D))
  e1F = _layer_b(h1F, e0F, s1, ss1, gamma1.reshape(1, D),
                 beta1.reshape(1, D))
  e1flat = e1F.reshape(4 * NP, Q)
  msg2A = _edge(srcA, dst2, e1flat, z16)
  msg2B = _edge(srcB, dst2, e1flat, z16)
  h2F, s2, ss2 = _layer_a(msg2A, msg2B, deg2, nodes_num_norm_sqrt, Wo,
                          bo.reshape(1, D))
  e2F = _layer_b(h2F, e1F, s2, ss2, gamma_o.reshape(1, D),
                 beta_o.reshape(1, D))
  hgsum, cnt = _pool(e1flat, e2F.reshape(4 * NP, Q), gidP, z16, z8, ones128)
  return _final(hgsum, cnt, W_out, b_out.reshape(1, D))


# speedup vs baseline: 6.1092x; 1.1158x over previous
import jax
import jax.numpy as jnp
from jax import lax
from jax.experimental import pallas as pl
from jax.experimental.pallas import tpu as pltpu
from jax.experimental.pallas import tpu_sc as plsc

N = 50000
E = 800000
G = 256
D = 64
Q = 16
EPS = 1e-5

NP = 50176
ROWS = 6272
EP = ROWS * 128
TROWS = ROWS // 16
GRP = 8
NGRP = TROWS // GRP
DEGSPLIT = 24
R = NP
ZCH = 784
NT = NP // 16
GACC = 1032
PROWS = 4 * NP // 128
BLK = 3136
GRID = NP // BLK

_mesh = plsc.VectorSubcoreMesh(core_axis_name="c", subcore_axis_name="s")
_sc_params = pltpu.CompilerParams(use_tc_tiling_on_sc=False)


def _make_edge():
  def body(src_hbm, dst_hbm, x_hbm, z16,
           msg_out,
           src_a, dst_a, rows_a, src_b, dst_b, rows_b,
           acc_sh, sem_a, sem_b):
    c = lax.axis_index("c")
    s = lax.axis_index("s")

    def fire(p, g, src_v, dst_v, rows_v, sem):
      r0 = s * TROWS + g * GRP
      pltpu.sync_copy(src_hbm.at[p, c, pl.ds(r0, GRP), :], src_v)
      pltpu.sync_copy(dst_hbm.at[pl.ds(r0, GRP), :], dst_v)
      for j in range(GRP):
        pltpu.async_copy(x_hbm.at[src_v.at[j]], rows_v.at[j], sem)

    def drain_scatter(src_v, dst_v, rows_v, sem):
      for j in range(GRP):
        pltpu.make_async_copy(x_hbm.at[src_v.at[j]], rows_v.at[j],
                              sem).wait()
        pltpu.sync_copy(rows_v.at[j], acc_sh.at[dst_v.at[j]], add=True)

    for p in range(2):
      base = s * NT
      for q in range(4):
        pltpu.sync_copy(z16, acc_sh.at[pl.ds(base + q * ZCH, ZCH), :])
      plsc.subcore_barrier()

      fire(p, 0, src_a, dst_a, rows_a, sem_a)

      @pl.loop(0, NGRP // 2)
      def _(k):
        fire(p, 2 * k + 1, src_b, dst_b, rows_b, sem_b)
        drain_scatter(src_a, dst_a, rows_a, sem_a)
        fire(p, 2 * k + 2, src_a, dst_a, rows_a, sem_a)
        drain_scatter(src_b, dst_b, rows_b, sem_b)

      drain_scatter(src_a, dst_a, rows_a, sem_a)

      plsc.subcore_barrier()
      pltpu.sync_copy(acc_sh.at[pl.ds(s * NT, NT), :],
                      msg_out.at[pl.ds(s * NT, NT), 2 * p + c, :])

  return pl.kernel(
      body,
      out_type=jax.ShapeDtypeStruct((NP, 4, Q), jnp.float32),
      mesh=_mesh,
      scratch_types=[
          pltpu.VMEM((GRP, 128), jnp.int32),
          pltpu.VMEM((GRP, 128), jnp.int32),
          pltpu.VMEM((GRP, 128, Q), jnp.float32),
          pltpu.VMEM((GRP, 128), jnp.int32),
          pltpu.VMEM((GRP, 128), jnp.int32),
          pltpu.VMEM((GRP, 128, Q), jnp.float32),
          pltpu.VMEM_SHARED((R, Q), jnp.float32),
          pltpu.SemaphoreType.DMA,
          pltpu.SemaphoreType.DMA,
      ],
      compiler_params=_sc_params)


_edge = _make_edge()


def _make_deg():
  def body(dst_hbm, z8, ones_hbm, deg_out, dst_v, ones_v, deg_sh):
    c = lax.axis_index("c")
    s = lax.axis_index("s")
    base = s * NT
    for q in range(4):
      pltpu.sync_copy(z8, deg_sh.at[pl.ds(base + q * ZCH, ZCH), :])
    pltpu.sync_copy(ones_hbm, ones_v)
    plsc.subcore_barrier()

    @pl.loop(0, NGRP)
    def _(g):
      @pl.when((g < DEGSPLIT) == (c == 0))
      def _():
        r0 = s * TROWS + g * GRP
        pltpu.sync_copy(dst_hbm.at[pl.ds(r0, GRP), :], dst_v)
        for j in range(GRP):
          pltpu.sync_copy(ones_v, deg_sh.at[dst_v.at[j]], add=True)

    plsc.subcore_barrier()
    pltpu.sync_copy(deg_sh.at[pl.ds(s * NT, NT), :],
                    deg_out.at[c, pl.ds(s * NT, NT), :])

  return pl.kernel(
      body,
      out_type=jax.ShapeDtypeStruct((2, NP, 8), jnp.float32),
      mesh=_mesh,
      scratch_types=[
          pltpu.VMEM((GRP, 128), jnp.int32),
          pltpu.VMEM((128, 8), jnp.float32),
          pltpu.VMEM_SHARED((R, 8), jnp.float32),
      ],
      compiler_params=_sc_params)


_deg = _make_deg()


def _make_pool():
  def body(e1_hbm, e2_hbm, gidx_hbm, z16, z8, ones_hbm,
           hg_out, cnt_out,
           gid_v, chunk_v, ones_v, gacc_sh, cnt_sh):
    c = lax.axis_index("c")
    s = lax.axis_index("s")
    pltpu.sync_copy(z16.at[pl.ds(0, 64), :], gacc_sh.at[pl.ds(s * 64, 64), :])
    pltpu.sync_copy(z8.at[pl.ds(0, 64), :], cnt_sh.at[pl.ds(s * 64, 64), :])
    @pl.when(s == 0)
    def _():
      pltpu.sync_copy(z16.at[pl.ds(64, 8), :], gacc_sh.at[pl.ds(1024, 8), :])
      pltpu.sync_copy(z8.at[pl.ds(64, 8), :], cnt_sh.at[pl.ds(1024, 8), :])
    pltpu.sync_copy(ones_hbm, ones_v)
    plsc.subcore_barrier()

    @pl.loop(0, PROWS // 32)
    def _(k):
      row = c * (PROWS // 2) + s * (PROWS // 32) + k
      pltpu.sync_copy(gidx_hbm.at[row], gid_v)
      pltpu.sync_copy(e1_hbm.at[pl.ds(row * 128, 128), :], chunk_v)
      pltpu.sync_copy(chunk_v, gacc_sh.at[gid_v], add=True)
      pltpu.sync_copy(e2_hbm.at[pl.ds(row * 128, 128), :], chunk_v)
      pltpu.sync_copy(chunk_v, gacc_sh.at[gid_v], add=True)
      pltpu.sync_copy(ones_v, cnt_sh.at[gid_v], add=True)

    plsc.subcore_barrier()
    pltpu.sync_copy(gacc_sh.at[pl.ds(s * 64, 64), :],
                    hg_out.at[c, pl.ds(s * 64, 64), :])
    @pl.when(s == 0)
    def _():
      pltpu.sync_copy(gacc_sh.at[pl.ds(1024, 8), :],
                      hg_out.at[c, pl.ds(1024, 8), :])
    pltpu.sync_copy(cnt_sh.at[pl.ds(s * 64, 64), :],
                    cnt_out.at[c, pl.ds(s * 64, 64), :])
    @pl.when(s == 0)
    def _():
      pltpu.sync_copy(cnt_sh.at[pl.ds(1024, 8), :],
                      cnt_out.at[c, pl.ds(1024, 8), :])

  return pl.kernel(
      body,
      out_type=[jax.ShapeDtypeStruct((2, GACC, Q), jnp.float32),
                jax.ShapeDtypeStruct((2, GACC, 8), jnp.float32)],
      mesh=_mesh,
      scratch_types=[
          pltpu.VMEM((128,), jnp.int32),
          pltpu.VMEM((128, Q), jnp.float32),
          pltpu.VMEM((128, 8), jnp.float32),
          pltpu.VMEM_SHARED((GACC, Q), jnp.float32),
          pltpu.VMEM_SHARED((GACC, 8), jnp.float32),
      ],
      compiler_params=_sc_params)


_pool = _make_pool()


def _emb_call(x, w, b, deg2):
  def body(x_ref, w_ref, b_ref, d0, d1, o_ref, inv_ref):
    y = jnp.dot(x_ref[...], w_ref[...],
                preferred_element_type=jnp.float32) + b_ref[...]
    o_ref[...] = y
    deg = d0[0, :, 0:1] + d1[0, :, 0:1]
    inv_ref[...] = (1.0 / jnp.maximum(deg, 1.0)) * jnp.ones(
        (1, D), jnp.float32)

  return pl.pallas_call(
      body,
      grid=(GRID,),
      in_specs=[pl.BlockSpec((BLK, D), lambda i: (i, 0)),
                pl.BlockSpec((D, D), lambda i: (0, 0)),
                pl.BlockSpec((1, D), lambda i: (0, 0)),
                pl.BlockSpec((1, BLK, 8), lambda i: (0, i, 0)),
                pl.BlockSpec((1, BLK, 8), lambda i: (1, i, 0))],
      out_specs=[pl.BlockSpec((BLK, D), lambda i: (i, 0)),
                 pl.BlockSpec((BLK, D), lambda i: (i, 0))],
      out_shape=[jax.ShapeDtypeStruct((NP, D), jnp.float32),
                 jax.ShapeDtypeStruct((NP, D), jnp.float32)],
  )(x, w, b, deg2, deg2)


def _layer_a(msg64, inv64, snorm, w, b):
  def body(m_ref, i_ref, sn, w_ref, b_ref, o_ref, s_ref, ss_ref):
    agg = m_ref[...] * i_ref[...]
    h = jnp.dot(agg, w_ref[...], preferred_element_type=jnp.float32)
    h = jnp.maximum(h + b_ref[...], 0.0) * sn[...]
    o_ref[...] = h

    @pl.when(pl.program_id(0) == 0)
    def _():
      s_ref[...] = jnp.zeros_like(s_ref)
      ss_ref[...] = jnp.zeros_like(ss_ref)

    rowid = (lax.broadcasted_iota(jnp.int32, (BLK, 1), 0)
             + pl.program_id(0) * BLK)
    hm = jnp.where(rowid < N, h, 0.0)
    s_ref[...] += jnp.sum(hm, axis=0, keepdims=True)
    ss_ref[...] += jnp.sum(hm * hm, axis=0, keepdims=True)

  return pl.pallas_call(
      body,
      grid=(GRID,),
      in_specs=[pl.BlockSpec((BLK, D), lambda i: (i, 0)),
                pl.BlockSpec((BLK, D), lambda i: (i, 0)),
                pl.BlockSpec((BLK, 1), lambda i: (i, 0)),
                pl.BlockSpec((D, D), lambda i: (0, 0)),
                pl.BlockSpec((1, D), lambda i: (0, 0))],
      out_specs=[pl.BlockSpec((BLK, D), lambda i: (i, 0)),
                 pl.BlockSpec((1, D), lambda i: (0, 0)),
                 pl.BlockSpec((1, D), lambda i: (0, 0))],
      out_shape=[jax.ShapeDtypeStruct((NP, D), jnp.float32),
                 jax.ShapeDtypeStruct((1, D), jnp.float32),
                 jax.ShapeDtypeStruct((1, D), jnp.float32)],
  )(msg64, inv64, snorm, w, b)


def _layer_b(h64, x64, ssum, sqsum, gamma, beta):
  def body(h_ref, x_ref, s_ref, ss_ref, g_ref, be_ref, o_ref):
    mu = s_ref[...] / N
    var = ss_ref[...] / N - mu * mu
    scale = g_ref[...] * lax.rsqrt(var + EPS)
    shift = be_ref[...] - mu * scale
    o_ref[...] = x_ref[...] + h_ref[...] * scale + shift

  return pl.pallas_call(
      body,
      grid=(GRID,),
      in_specs=[pl.BlockSpec((BLK, D), lambda i: (i, 0)),
                pl.BlockSpec((BLK, D), lambda i: (i, 0)),
                pl.BlockSpec((1, D), lambda i: (0, 0)),
                pl.BlockSpec((1, D), lambda i: (0, 0)),
                pl.BlockSpec((1, D), lambda i: (0, 0)),
                pl.BlockSpec((1, D), lambda i: (0, 0))],
      out_specs=pl.BlockSpec((BLK, D), lambda i: (i, 0)),
      out_shape=jax.ShapeDtypeStruct((NP, D), jnp.float32),
  )(h64, x64, ssum, sqsum, gamma, beta)


def _final(hg4, cnt4, w, b):
  def body(p00, p01, p02, p03, p10, p11, p12, p13, c0, c1, w_ref, b_ref,
           o_ref):
    cnt = c0[0, :, 0:1] + c1[0, :, 0:1]
    inv = 1.0 / jnp.maximum(cnt, 1.0)
    o = b_ref[...] * jnp.ones((G, 1), jnp.float32)
    p0 = (p00, p01, p02, p03)
    p1 = (p10, p11, p12, p13)
    for q in range(4):
      sq = (p0[q][0] + p1[q][0]) * inv
      o = o + jnp.dot(sq, w_ref[q * Q:(q + 1) * Q, :],
                      preferred_element_type=jnp.float32)
    o_ref[...] = o

  qspecs = [pl.BlockSpec((1, G, Q), lambda i, c=c, q=q: (c, q, 0))
            for c in range(2) for q in range(4)]
  return pl.pallas_call(
      body,
      grid=(1,),
      in_specs=qspecs +
      [pl.BlockSpec((1, G, 8), lambda i: (0, 0, 0)),
       pl.BlockSpec((1, G, 8), lambda i: (1, 0, 0)),
       pl.BlockSpec((D, D), lambda i: (0, 0)),
       pl.BlockSpec((1, D), lambda i: (0, 0))],
      out_specs=pl.BlockSpec((G, D), lambda i: (0, 0)),
      out_shape=jax.ShapeDtypeStruct((G, D), jnp.float32),
  )(*([hg4] * 8), cnt4, cnt4, w, b)


def kernel(nodes_feat, edge_index, edges_feat, nodes_num_norm_sqrt,
           edges_num_norm_sqrt, graph_ids, W_emb, b_emb, W1, b1, gamma1,
           beta1, Wo, bo, gamma_o, beta_o, W_out, b_out):
  src = edge_index[0]
  dst = edge_index[1]
  pad = EP - E
  srcP = 4 * jnp.concatenate([src, jnp.zeros((pad,), jnp.int32)])
  dstP = jnp.concatenate([dst, jnp.full((pad,), N, jnp.int32)])
  src4 = jnp.stack([srcP, srcP + 1, srcP + 2,
                    srcP + 3]).reshape(2, 2, ROWS, 128)
  dst2 = dstP.reshape(ROWS, 128)
  gidQ = 256 * jnp.tile(jnp.arange(4, dtype=jnp.int32), NP)
  gidN = jnp.repeat(
      jnp.concatenate([graph_ids, jnp.full((NP - N,), -1, jnp.int32)]), 4)
  gidx = jnp.where(gidN >= 0, gidQ + gidN,
                   1024 + jnp.tile(jnp.arange(4, dtype=jnp.int32), NP))
  gidx2 = gidx.reshape(PROWS, 128)
  z16 = jnp.zeros((ZCH, Q), jnp.float32)
  z8 = jnp.zeros((ZCH, 8), jnp.float32)
  ones128 = jnp.ones((128, 8), jnp.float32)

  deg2 = _deg(dst2, z8, ones128)
  e0, inv64 = _emb_call(nodes_feat, W_emb, b_emb.reshape(1, D), deg2)
  msg1 = _edge(src4, dst2, e0.reshape(4 * NP, Q), z16).reshape(NP, D)
  h1, s1, ss1 = _layer_a(msg1, inv64, nodes_num_norm_sqrt, W1,
                         b1.reshape(1, D))
  e1 = _layer_b(h1, e0, s1, ss1, gamma1.reshape(1, D), beta1.reshape(1, D))
  e1v = e1.reshape(4 * NP, Q)
  msg2 = _edge(src4, dst2, e1v, z16).reshape(NP, D)
  h2, s2, ss2 = _layer_a(msg2, inv64, nodes_num_norm_sqrt, Wo,
                         bo.reshape(1, D))
  e2 = _layer_b(h2, e1, s2, ss2, gamma_o.reshape(1, D),
                beta_o.reshape(1, D))
  hg4, cnt4 = _pool(e1v, e2.reshape(4 * NP, Q), gidx2, z16, z8, ones128)
  return _final(hg4, cnt4, W_out, b_out.reshape(1, D))
